# baseline (speedup 1.0000x reference)
import jax
import jax.numpy as jnp
from jax import lax
from jax.experimental import pallas as pl
from jax.experimental.pallas import tpu as pltpu
from jax.experimental.pallas import tpu_sc as plsc

BATCH = 16384
EMB_DIM = 64
NC = 2
NS = 16
NW = NC * NS
B_PER_W = BATCH // NW
CHUNK = 128
N_CHUNKS = B_PER_W // CHUNK


def _lookup_body(uid_idx_hbm, iid_idx_hbm, uid_table_hbm, iid_table_hbm,
                 uid_out_hbm, iid_out_hbm,
                 idx_u, idx_i, rows_u, rows_i, sem_u, sem_i, sem_out):
    wid = lax.axis_index("s") * NC + lax.axis_index("c")
    base = wid * B_PER_W
    pltpu.sync_copy(uid_idx_hbm.at[pl.ds(base, B_PER_W)], idx_u)
    pltpu.sync_copy(iid_idx_hbm.at[pl.ds(base, B_PER_W)], idx_i)
    copies = []
    for j in range(N_CHUNKS):
        sl = pl.ds(j * CHUNK, CHUNK)
        copies.append(pltpu.async_copy(
            uid_table_hbm.at[idx_u.at[sl]], rows_u.at[sl], sem_u))
        copies.append(pltpu.async_copy(
            iid_table_hbm.at[idx_i.at[sl]], rows_i.at[sl], sem_i))
    for c in copies:
        c.wait()
    out_sl = pl.ds(base, B_PER_W)
    cu = pltpu.async_copy(rows_u, uid_out_hbm.at[out_sl], sem_out)
    ci = pltpu.async_copy(rows_i, iid_out_hbm.at[out_sl], sem_out)
    cu.wait()
    ci.wait()


def kernel(x, uid_table, iid_table):
    uid_idx = x[:, 0]
    iid_idx = x[:, 1]
    mesh = plsc.VectorSubcoreMesh(core_axis_name="c", subcore_axis_name="s")
    f = pl.kernel(
        _lookup_body,
        out_type=(
            jax.ShapeDtypeStruct((BATCH, EMB_DIM), jnp.float32),
            jax.ShapeDtypeStruct((BATCH, EMB_DIM), jnp.float32),
        ),
        mesh=mesh,
        scratch_types=[
            pltpu.VMEM((B_PER_W,), jnp.int32),
            pltpu.VMEM((B_PER_W,), jnp.int32),
            pltpu.VMEM((B_PER_W, EMB_DIM), jnp.float32),
            pltpu.VMEM((B_PER_W, EMB_DIM), jnp.float32),
            pltpu.SemaphoreType.DMA,
            pltpu.SemaphoreType.DMA,
            pltpu.SemaphoreType.DMA,
        ],
        compiler_params=pltpu.CompilerParams(use_tc_tiling_on_sc=False),
    )
    return f(uid_idx, iid_idx, uid_table, iid_table)
